# baseline (speedup 1.0000x reference)
import numpy as np
import jax
import jax.numpy as jnp
from jax import lax
from jax.experimental import pallas as pl
from jax.experimental.pallas import tpu as pltpu
from jax.experimental.pallas import tpu_sc as plsc

N = 1_000_000
B = 128
K = 256
NW = 16
NV = 3912
CH = NV * 16
PADN = CH * NW
SELCAP = 272
TOPBIT = np.uint32(0x80000000)


def _keys_of(x):
    b = lax.bitcast_convert_type(x, jnp.uint32)
    neg = b >= TOPBIT
    flip = jnp.where(neg, np.uint32(0xFFFFFFFF), TOPBIT)
    return b ^ flip


def _sload(ref, *idx):
    idxs = [jnp.broadcast_to(i, (16,)) for i in idx]
    return plsc.load_gather(ref, idxs)[0]


def _body(scores_hbm, labels_hbm, out_hbm, lab_v, hist_v, lhist_v, merge_v,
          sel_v, sel_i, tie_v, tie_i, gidx_v, gdat_v, info_v, red_v,
          outacc_v, sh_hist, sh_info, sh_acc, sem):
    wid = lax.axis_index("s")
    lanes = lax.iota(jnp.int32, 16)
    ones16 = jnp.ones((16,), jnp.int32)

    pltpu.sync_copy(labels_hbm.at[pl.ds(wid * NV, NV)], lab_v)

    prefix = np.uint32(0)
    r = np.int32(K)
    for lvl in range(4):
        sh = np.uint32(24 - 8 * lvl)
        psh = np.uint32(32 - 8 * lvl)

        def zbody(i, c):
            plsc.store_scatter(hist_v, [i * 16 + lanes],
                               jnp.zeros((16,), jnp.int32))
            return c
        lax.fori_loop(0, 256, zbody, 0)

        def sbody(i, c, _lvl=lvl, _sh=sh, _psh=psh, _prefix=prefix):
            ku = _keys_of(lab_v[i])
            binv = lax.convert_element_type(
                (ku >> _sh) & np.uint32(0xFF), jnp.int32)
            slot = binv * 16 + lanes
            if _lvl == 0:
                plsc.addupdate_scatter(hist_v, [slot], ones16)
            else:
                msk = (ku >> _psh) == _prefix
                plsc.addupdate_scatter(hist_v, [slot], ones16, mask=msk)
            return c
        lax.fori_loop(0, NV, sbody, 0)

        def rbody(g, c):
            def jbody(j, acc):
                return acc + plsc.load_gather(
                    hist_v, [g * 256 + lanes * 16 + j])
            red = lax.fori_loop(0, 16, jbody, jnp.zeros((16,), jnp.int32))
            plsc.store_scatter(lhist_v, [g * 16 + lanes], red)
            return c
        lax.fori_loop(0, 16, rbody, 0)

        pltpu.sync_copy(lhist_v, sh_hist.at[pl.ds(wid * 256, 256)])
        plsc.subcore_barrier()
        pltpu.sync_copy(sh_hist, merge_v)
        plsc.subcore_barrier()

        def gbody(g, c):
            def wbody(w, acc):
                return acc + merge_v[pl.ds(w * 256 + g * 16, 16)]
            red = lax.fori_loop(0, 16, wbody, jnp.zeros((16,), jnp.int32))
            plsc.store_scatter(lhist_v, [g * 16 + lanes], red)
            return c
        lax.fori_loop(0, 16, gbody, 0)

        def cond(c):
            return c[1] < r

        def step(c):
            b2 = c[0] - 1
            return (b2, c[1] + _sload(lhist_v, b2))
        t, s_at_t = lax.while_loop(cond, step, (np.int32(256), np.int32(0)))
        c_t = _sload(lhist_v, t)
        r = r - (s_at_t - c_t)
        prefix = (prefix << np.uint32(8)) | lax.convert_element_type(t, jnp.uint32)

    T = prefix
    base = wid * np.int32(CH)

    def fbody(i, carry):
        nsel, ncap, ntot = carry
        x = lab_v[i]
        ku = _keys_of(x)
        idxv = base + i * 16 + lanes
        gt = ku > T
        gti = lax.convert_element_type(gt, jnp.int32)
        rg = plsc.cumsum(gti)
        posg = nsel + rg - gti
        plsc.store_scatter(sel_v, [posg], x, mask=gt)
        plsc.store_scatter(sel_i, [posg], idxv, mask=gt)
        cg = jnp.max(rg)
        eq = ku == T
        eqi = lax.convert_element_type(eq, jnp.int32)
        re = plsc.cumsum(eqi)
        keep = eq & ((ncap + re) <= np.int32(K))
        pose = ncap + re - eqi
        plsc.store_scatter(tie_v, [pose], x, mask=keep)
        plsc.store_scatter(tie_i, [pose], idxv, mask=keep)
        ce = jnp.max(re)
        kc = jnp.minimum(ce, jnp.maximum(np.int32(K) - ncap, 0))
        return (nsel + cg, ncap + kc, ntot + ce)

    nsel, ncap, ntot = lax.fori_loop(
        0, NV, fbody, (np.int32(0), np.int32(0), np.int32(0)))

    info_v[...] = jnp.where(lanes == 0, ntot, np.int32(0))
    pltpu.sync_copy(info_v, sh_info.at[pl.ds(wid * 16, 16)])
    plsc.subcore_barrier()
    pltpu.sync_copy(sh_info, merge_v.at[pl.ds(0, 256)])

    def pbody(w, p):
        return p + jnp.where(w < wid, _sload(merge_v, w * 16), np.int32(0))
    ptie = lax.fori_loop(0, 16, pbody, np.int32(0))
    q = jnp.clip(r - ptie, np.int32(0), ntot)
    ntake = nsel + q

    ramp = lanes * np.int32(N)

    def gatherbody(e, acc):
        is_sel = e < nsel
        e2 = jnp.maximum(e - nsel, 0)
        val = jnp.where(is_sel, _sload(sel_v, e), _sload(tie_v, e2))
        id0 = jnp.where(is_sel, _sload(sel_i, e), _sload(tie_i, e2))
        for v in range(8):
            gidx_v[pl.ds(v * 16, 16)] = id0 + np.int32(v * 16 * N) + ramp
        pltpu.async_copy(scores_hbm.at[gidx_v], gdat_v, sem).wait()
        vv = jnp.broadcast_to(val, (16,))
        for v in range(8):
            d = gdat_v[pl.ds(v * 16, 16)] - vv
            acc = acc + d * d
        return acc

    acc = lax.fori_loop(0, ntake, gatherbody, jnp.zeros((16,), jnp.float32))

    outacc_v[...] = acc
    pltpu.sync_copy(outacc_v, sh_acc.at[pl.ds(wid * 16, 16)])
    plsc.subcore_barrier()

    @pl.when(wid == 0)
    def _():
        pltpu.sync_copy(sh_acc, red_v)

        def rb(w, a):
            return a + red_v[pl.ds(w * 16, 16)]
        tot = lax.fori_loop(0, 16, rb, jnp.zeros((16,), jnp.float32))
        loss = jnp.sum(tot) * np.float32(1.0 / (B * K))
        outacc_v[...] = jnp.broadcast_to(loss, (16,))
        pltpu.sync_copy(outacc_v, out_hbm)


def _sc_call(scores_flat, labels2d):
    mesh = plsc.VectorSubcoreMesh(
        core_axis_name="c", subcore_axis_name="s", num_cores=1, num_subcores=16)
    kfn = pl.kernel(
        _body,
        out_type=jax.ShapeDtypeStruct((16,), jnp.float32),
        mesh=mesh,
        compiler_params=pltpu.CompilerParams(
            needs_layout_passes=False, use_tc_tiling_on_sc=False),
        scratch_types=[
            pltpu.VMEM((NV, 16), jnp.float32),
            pltpu.VMEM((4096,), jnp.int32),
            pltpu.VMEM((256,), jnp.int32),
            pltpu.VMEM((4096,), jnp.int32),
            pltpu.VMEM((SELCAP,), jnp.float32),
            pltpu.VMEM((SELCAP,), jnp.int32),
            pltpu.VMEM((SELCAP,), jnp.float32),
            pltpu.VMEM((SELCAP,), jnp.int32),
            pltpu.VMEM((B,), jnp.int32),
            pltpu.VMEM((B,), jnp.float32),
            pltpu.VMEM((16,), jnp.int32),
            pltpu.VMEM((256,), jnp.float32),
            pltpu.VMEM((16,), jnp.float32),
            pltpu.VMEM_SHARED((4096,), jnp.int32),
            pltpu.VMEM_SHARED((256,), jnp.int32),
            pltpu.VMEM_SHARED((256,), jnp.float32),
            pltpu.SemaphoreType.DMA,
        ],
    )
    return kfn(scores_flat, labels2d)


@jax.jit
def kernel(scores, labels):
    scores_flat = scores.reshape(B * N)
    lab = labels.reshape(N)
    pad = jnp.full((PADN - N,), -jnp.inf, jnp.float32)
    lab2d = jnp.concatenate([lab, pad]).reshape(PADN // 16, 16)
    out = _sc_call(scores_flat, lab2d)
    return out[0]
